# baseline (speedup 1.0000x reference)
import functools

import jax
import jax.numpy as jnp
from jax import lax
from jax.experimental import pallas as pl
from jax.experimental.pallas import tpu as pltpu
from jax.experimental.pallas import tpu_sc as plsc

L = 16
NC = 2
NS = 16
NW = NC * NS
ROWS_OUT = 16384
D = 128
B_W = ROWS_OUT // NW
CH = 128
NCH = B_W // CH


def _gather_body(x_hbm, out_hbm, idx_v, rows_v, sem):
    wid = lax.axis_index("s") * NC + lax.axis_index("c")
    base = wid * B_W

    def chunk(c, carry):
        start = base + c * CH

        def lanes(t, carry2):
            r = start + t * L + lax.iota(jnp.int32, L)
            src = 100 * (r >> 1) + 51 + (r & 1)
            idx_v[pl.ds(t * L, L)] = src
            return carry2

        lax.fori_loop(0, CH // L, lanes, 0, unroll=True)
        pltpu.async_copy(x_hbm.at[idx_v], rows_v, sem).wait()
        pltpu.sync_copy(rows_v, out_hbm.at[pl.ds(start, CH)])
        return carry

    lax.fori_loop(0, NCH, chunk, 0)


@jax.jit
def _run(xr):
    mesh = plsc.VectorSubcoreMesh(core_axis_name="c", subcore_axis_name="s")
    return pl.kernel(
        _gather_body,
        out_type=jax.ShapeDtypeStruct((ROWS_OUT, D), jnp.float32),
        mesh=mesh,
        scratch_types=[
            pltpu.VMEM((CH,), jnp.int32),
            pltpu.VMEM((CH, D), jnp.float32),
            pltpu.SemaphoreType.DMA,
        ],
    )(xr)


def kernel(x):
    xr = x.reshape(ROWS_OUT * 50, D)
    out = _run(xr)
    return out.reshape(ROWS_OUT // 2, 2, D)


# speedup vs baseline: 26.5362x; 26.5362x over previous
import functools

import jax
import jax.numpy as jnp
from jax import lax
from jax.experimental import pallas as pl
from jax.experimental.pallas import tpu as pltpu
from jax.experimental.pallas import tpu_sc as plsc

L = 16
NC = 2
NS = 16
NW = NC * NS
ROWS_OUT = 16384
D = 128
B_W = ROWS_OUT // NW
CH = 128
NCH = B_W // CH


def _gather_body(x_hbm, out_hbm, idx_v, rows_v, sem):
    wid = lax.axis_index("s") * NC + lax.axis_index("c")
    base = wid * B_W

    def chunk(c, carry):
        start = base + c * CH

        def lanes(t, carry2):
            r = start + t * L + lax.iota(jnp.int32, L)
            src = ((r & 1) + 1) * 16384 + ((r >> 1) << 1) + 1
            idx_v[pl.ds(t * L, L)] = src
            return carry2

        lax.fori_loop(0, CH // L, lanes, 0, unroll=True)
        pltpu.async_copy(x_hbm.at[idx_v], rows_v, sem).wait()
        pltpu.sync_copy(rows_v, out_hbm.at[pl.ds(start, CH)])
        return carry

    lax.fori_loop(0, NCH, chunk, 0)


@jax.jit
def _run(xr):
    mesh = plsc.VectorSubcoreMesh(core_axis_name="c", subcore_axis_name="s")
    return pl.kernel(
        _gather_body,
        out_type=jax.ShapeDtypeStruct((ROWS_OUT, D), jnp.float32),
        mesh=mesh,
        scratch_types=[
            pltpu.VMEM((CH,), jnp.int32),
            pltpu.VMEM((CH, D), jnp.float32),
            pltpu.SemaphoreType.DMA,
        ],
    )(xr)


def kernel(x):
    xr = jnp.transpose(x, (1, 0, 2)).reshape(50 * ROWS_OUT, D)
    out = _run(xr)
    return out.reshape(ROWS_OUT // 2, 2, D)


# speedup vs baseline: 28.7694x; 1.0842x over previous
import functools

import jax
import jax.numpy as jnp
from jax import lax
from jax.experimental import pallas as pl
from jax.experimental.pallas import tpu as pltpu
from jax.experimental.pallas import tpu_sc as plsc

L = 16
NC = 2
NS = 16
NW = NC * NS
ROWS_OUT = 16384
D = 128
B_W = ROWS_OUT // NW
CH = 128
NCH = B_W // CH


def _gather_body(x_hbm, out_hbm, idx_v, rows_v, gsem, ssem):
    wid = lax.axis_index("s") * NC + lax.axis_index("c")
    base = wid * B_W

    for c in range(NCH):
        for t in range(CH // L):
            r = base + c * CH + t * L + lax.iota(jnp.int32, L)
            src = ((r & 1) + 1) * 16384 + ((r >> 1) << 1) + 1
            idx_v[c, pl.ds(t * L, L)] = src

    gathers = [
        pltpu.async_copy(x_hbm.at[idx_v.at[c]], rows_v.at[c], gsem)
        for c in range(NCH)
    ]
    scatters = []
    for c in range(NCH):
        gathers[c].wait()
        scatters.append(
            pltpu.async_copy(
                rows_v.at[c], out_hbm.at[pl.ds(base + c * CH, CH)], ssem
            )
        )
    for s in scatters:
        s.wait()


@jax.jit
def _run(xr):
    mesh = plsc.VectorSubcoreMesh(core_axis_name="c", subcore_axis_name="s")
    return pl.kernel(
        _gather_body,
        out_type=jax.ShapeDtypeStruct((ROWS_OUT, D), jnp.float32),
        mesh=mesh,
        scratch_types=[
            pltpu.VMEM((NCH, CH), jnp.int32),
            pltpu.VMEM((NCH, CH, D), jnp.float32),
            pltpu.SemaphoreType.DMA,
            pltpu.SemaphoreType.DMA,
        ],
    )(xr)


def kernel(x):
    xr = jnp.transpose(x, (1, 0, 2)).reshape(50 * ROWS_OUT, D)
    out = _run(xr)
    return out.reshape(ROWS_OUT // 2, 2, D)
